# baseline (speedup 1.0000x reference)
import functools

import jax
import jax.numpy as jnp
from jax import lax
from jax.experimental import pallas as pl
from jax.experimental.pallas import tpu as pltpu
from jax.experimental.pallas import tpu_sc as plsc

N = 10000
E = 320000
D = 128

NC = 2
NS = 16
NW = NC * NS
C = 128
N_PAD = 10240
EP = 10240
NCH = EP // C
HNCH = NCH // 2
E_PAD = NW * EP
ROWS_PT = N_PAD // NS

_mesh = plsc.VectorSubcoreMesh(core_axis_name="c", subcore_axis_name="s")


@functools.partial(
    pl.kernel,
    out_type=jax.ShapeDtypeStruct((NC, 2, N_PAD), jnp.float32),
    mesh=_mesh,
    scratch_types=[
        pltpu.VMEM((NCH, C), jnp.int32),
        pltpu.VMEM((NCH, C), jnp.int32),
        pltpu.VMEM((C,), jnp.float32),
        pltpu.VMEM_SHARED((N_PAD,), jnp.float32),
        pltpu.VMEM_SHARED((N_PAD,), jnp.float32),
    ],
)
def _deg_kernel(src_hbm, dst_hbm, ones_hbm, zeros_hbm, out_hbm,
                sidx, didx, ones_v, acc_out, acc_in):
    cid = lax.axis_index("c")
    sid = lax.axis_index("s")
    wid = cid * NS + sid
    zslice = pl.ds(sid * (N_PAD // NS), N_PAD // NS)
    pltpu.sync_copy(zeros_hbm.at[zslice], acc_out.at[zslice])
    pltpu.sync_copy(zeros_hbm.at[zslice], acc_in.at[zslice])
    pltpu.sync_copy(src_hbm.at[wid], sidx)
    pltpu.sync_copy(dst_hbm.at[wid], didx)
    pltpu.sync_copy(ones_hbm, ones_v)
    plsc.subcore_barrier()

    def body(j, carry):
        pltpu.sync_copy(ones_v, acc_out.at[sidx.at[j]], add=True)
        pltpu.sync_copy(ones_v, acc_in.at[didx.at[j]], add=True)
        return carry

    lax.fori_loop(0, NCH, body, 0)
    plsc.subcore_barrier()
    pltpu.sync_copy(acc_out.at[zslice], out_hbm.at[cid, 0, zslice])
    pltpu.sync_copy(acc_in.at[zslice], out_hbm.at[cid, 1, zslice])


@functools.partial(
    pl.kernel,
    out_type=jax.ShapeDtypeStruct((NC, N_PAD, D), jnp.float32),
    mesh=_mesh,
    scratch_types=[
        pltpu.VMEM((HNCH, C), jnp.int32),
        pltpu.VMEM((HNCH, C), jnp.int32),
        pltpu.VMEM((C, D), jnp.float32),
        pltpu.VMEM((C, D), jnp.float32),
        pltpu.VMEM_SHARED((N_PAD, D), jnp.float32),
        pltpu.SemaphoreType.DMA,
        pltpu.SemaphoreType.DMA,
        pltpu.SemaphoreType.DMA,
        pltpu.SemaphoreType.DMA,
    ],
)
def _conv_kernel(t_hbm, src_hbm, dst_hbm, zeros_hbm, out_hbm,
                 sidx, didx, rows0, rows1, acc, gsem0, gsem1, ssem0, ssem1):
    cid = lax.axis_index("c")
    sid = lax.axis_index("s")
    wid = cid * NS + sid
    zslice = pl.ds(sid * ROWS_PT, ROWS_PT)
    pltpu.async_copy(zeros_hbm.at[zslice], acc.at[zslice], gsem0)
    pltpu.async_copy(src_hbm.at[wid, pl.ds(0, HNCH)], sidx, gsem1)
    pltpu.async_copy(dst_hbm.at[wid, pl.ds(0, HNCH)], didx, gsem1)
    pltpu.make_async_copy(zeros_hbm.at[zslice], acc.at[zslice], gsem0).wait()
    pltpu.make_async_copy(src_hbm.at[wid, pl.ds(0, HNCH)], sidx, gsem1).wait()
    pltpu.make_async_copy(dst_hbm.at[wid, pl.ds(0, HNCH)], didx, gsem1).wait()
    plsc.subcore_barrier()

    def body(jj, carry):
        j0 = 2 * jj
        pltpu.make_async_copy(t_hbm.at[sidx.at[j0]], rows0, gsem0).wait()
        s0 = pltpu.async_copy(rows0, acc.at[didx.at[j0]], ssem0, add=True)
        pltpu.make_async_copy(t_hbm.at[sidx.at[j0 + 1]], rows1, gsem1).wait()
        s1 = pltpu.async_copy(rows1, acc.at[didx.at[j0 + 1]], ssem1, add=True)
        s0.wait()

        @pl.when(jj < HNCH // 2 - 1)
        def _():
            pltpu.async_copy(t_hbm.at[sidx.at[j0 + 2]], rows0, gsem0)

        s1.wait()

        @pl.when(jj < HNCH // 2 - 1)
        def _():
            pltpu.async_copy(t_hbm.at[sidx.at[j0 + 3]], rows1, gsem1)

        return carry

    for half in range(2):
        if half == 1:
            pltpu.sync_copy(src_hbm.at[wid, pl.ds(HNCH, HNCH)], sidx)
            pltpu.sync_copy(dst_hbm.at[wid, pl.ds(HNCH, HNCH)], didx)
        pltpu.async_copy(t_hbm.at[sidx.at[0]], rows0, gsem0)
        pltpu.async_copy(t_hbm.at[sidx.at[1]], rows1, gsem1)
        lax.fori_loop(0, HNCH // 2, body, 0)

    plsc.subcore_barrier()
    pltpu.sync_copy(acc.at[zslice], out_hbm.at[cid, zslice])


def _tc_mm_body(h_ref, w0_ref, q_ref):
    q_ref[...] = jnp.dot(h_ref[...], w0_ref[...],
                         preferred_element_type=jnp.float32)


def _tc1_body(deg_ref, q_ref, csrc_ref, cdst_ref, t0_ref):
    deg_out = deg_ref[0, 0, :N, :] + deg_ref[1, 0, :N, :]
    deg_in = deg_ref[0, 1, :N, :] + deg_ref[1, 1, :N, :]
    c_src = lax.rsqrt(jnp.maximum(deg_out, 1.0))
    c_dst = lax.rsqrt(jnp.maximum(deg_in, 1.0))
    csrc_ref[...] = c_src
    cdst_ref[...] = c_dst
    t0_ref[:N, :] = q_ref[...] * c_src
    t0_ref[N:, :] = jnp.zeros((N_PAD - N, D), jnp.float32)


def _tc_mid_body(p_ref, cdst_ref, b_ref, csrc_ref, w_ref, t_ref):
    agg = (p_ref[0, :N] + p_ref[1, :N]) * cdst_ref[...]
    hnew = jnp.maximum(agg + b_ref[...], 0.0)
    t_ref[:N, :] = jnp.dot(hnew * csrc_ref[...], w_ref[...],
                           preferred_element_type=jnp.float32)
    t_ref[N:, :] = jnp.zeros((N_PAD - N, D), jnp.float32)


def _tc_post_body(p_ref, cdst_ref, b_ref, out_ref):
    agg = (p_ref[0, :N] + p_ref[1, :N]) * cdst_ref[...]
    out_ref[...] = jnp.maximum(agg + b_ref[...], 0.0)


_tc_mm = pl.pallas_call(
    _tc_mm_body,
    out_shape=jax.ShapeDtypeStruct((N, D), jnp.float32),
)

_tc1 = pl.pallas_call(
    _tc1_body,
    out_shape=(
        jax.ShapeDtypeStruct((N, 1), jnp.float32),
        jax.ShapeDtypeStruct((N, 1), jnp.float32),
        jax.ShapeDtypeStruct((N_PAD, D), jnp.float32),
    ),
)

_tc_mid = pl.pallas_call(
    _tc_mid_body,
    out_shape=jax.ShapeDtypeStruct((N_PAD, D), jnp.float32),
)

_tc_post = pl.pallas_call(
    _tc_post_body,
    out_shape=jax.ShapeDtypeStruct((N, D), jnp.float32),
)


def kernel(h, edge_index, W0, b0, W1, b1):
    pad = (jnp.arange(E_PAD - E, dtype=jnp.int32) % (N_PAD - N)) + N
    src = jnp.concatenate([edge_index[0].astype(jnp.int32), pad])
    dst = jnp.concatenate([edge_index[1].astype(jnp.int32), pad])
    src = src.reshape(NW, NCH, C)
    dst = dst.reshape(NW, NCH, C)
    ones_c = jnp.ones((C,), jnp.float32)
    zeros_pad = jnp.zeros((N_PAD,), jnp.float32)
    zeros_nd = jnp.zeros((N_PAD, D), jnp.float32)

    q0 = _tc_mm(h, W0)
    deg = _deg_kernel(src, dst, ones_c, zeros_pad)
    deg4 = deg.reshape(NC, 2, N_PAD, 1)
    c_src, c_dst, t0 = _tc1(deg4, q0)

    p0 = _conv_kernel(t0, src, dst, zeros_nd)
    t1 = _tc_mid(p0, c_dst, b0.reshape(1, D), c_src, W1)

    p1 = _conv_kernel(t1, src, dst, zeros_nd)
    return _tc_post(p1, c_dst, b1.reshape(1, D))


# speedup vs baseline: 1.2264x; 1.2264x over previous
import functools

import jax
import jax.numpy as jnp
from jax import lax
from jax.experimental import pallas as pl
from jax.experimental.pallas import tpu as pltpu
from jax.experimental.pallas import tpu_sc as plsc

N = 10000
E = 320000
D = 128

NC = 2
NS = 16
NW = NC * NS
C = 128
N_PAD = 10240
EP = 10240
NCH = EP // C
HNCH = NCH // 2
E_PAD = NW * EP
ROWS_PT = N_PAD // NS

_mesh = plsc.VectorSubcoreMesh(core_axis_name="c", subcore_axis_name="s")


@functools.partial(
    pl.kernel,
    out_type=jax.ShapeDtypeStruct((NC, 2, N_PAD), jnp.float32),
    mesh=_mesh,
    scratch_types=[
        pltpu.VMEM((NCH, C), jnp.int32),
        pltpu.VMEM((NCH, C), jnp.int32),
        pltpu.VMEM((C,), jnp.float32),
        pltpu.VMEM_SHARED((N_PAD,), jnp.float32),
        pltpu.VMEM_SHARED((N_PAD,), jnp.float32),
    ],
)
def _deg_kernel(src_hbm, dst_hbm, ones_hbm, zeros_hbm, out_hbm,
                sidx, didx, ones_v, acc_out, acc_in):
    cid = lax.axis_index("c")
    sid = lax.axis_index("s")
    wid = cid * NS + sid
    zslice = pl.ds(sid * (N_PAD // NS), N_PAD // NS)
    pltpu.sync_copy(zeros_hbm.at[zslice], acc_out.at[zslice])
    pltpu.sync_copy(zeros_hbm.at[zslice], acc_in.at[zslice])
    pltpu.sync_copy(src_hbm.at[wid], sidx)
    pltpu.sync_copy(dst_hbm.at[wid], didx)
    pltpu.sync_copy(ones_hbm, ones_v)
    plsc.subcore_barrier()

    def body(j, carry):
        pltpu.sync_copy(ones_v, acc_out.at[sidx.at[j]], add=True)
        pltpu.sync_copy(ones_v, acc_in.at[didx.at[j]], add=True)
        return carry

    lax.fori_loop(0, NCH, body, 0)
    plsc.subcore_barrier()
    pltpu.sync_copy(acc_out.at[zslice], out_hbm.at[cid, 0, zslice])
    pltpu.sync_copy(acc_in.at[zslice], out_hbm.at[cid, 1, zslice])


@functools.partial(
    pl.kernel,
    out_type=jax.ShapeDtypeStruct((NC, N_PAD, D), jnp.float32),
    mesh=_mesh,
    scratch_types=[
        pltpu.VMEM((HNCH, C), jnp.int32),
        pltpu.VMEM((HNCH, C), jnp.int32),
        pltpu.VMEM((C, D), jnp.float32),
        pltpu.VMEM((C, D), jnp.float32),
        pltpu.VMEM_SHARED((N_PAD, D), jnp.float32),
        pltpu.SemaphoreType.DMA,
        pltpu.SemaphoreType.DMA,
        pltpu.SemaphoreType.DMA,
        pltpu.SemaphoreType.DMA,
    ],
)
def _conv_kernel(t_hbm, src_hbm, dst_hbm, zeros_hbm, out_hbm,
                 sidx, didx, rows0, rows1, acc, gsem0, gsem1, ssem0, ssem1):
    cid = lax.axis_index("c")
    sid = lax.axis_index("s")
    wid = cid * NS + sid
    zslice = pl.ds(sid * ROWS_PT, ROWS_PT)
    pltpu.async_copy(zeros_hbm.at[zslice], acc.at[zslice], gsem0)
    pltpu.async_copy(src_hbm.at[wid, pl.ds(0, HNCH)], sidx, gsem1)
    pltpu.async_copy(dst_hbm.at[wid, pl.ds(0, HNCH)], didx, gsem1)
    pltpu.make_async_copy(zeros_hbm.at[zslice], acc.at[zslice], gsem0).wait()
    pltpu.make_async_copy(src_hbm.at[wid, pl.ds(0, HNCH)], sidx, gsem1).wait()
    pltpu.make_async_copy(dst_hbm.at[wid, pl.ds(0, HNCH)], didx, gsem1).wait()
    plsc.subcore_barrier()

    def body(jj, carry):
        j0 = 2 * jj
        pltpu.async_copy(t_hbm.at[sidx.at[j0 + 1]], rows1, gsem1)
        pltpu.make_async_copy(t_hbm.at[sidx.at[j0]], rows0, gsem0).wait()
        pltpu.sync_copy(rows0, acc.at[didx.at[j0]], add=True)

        @pl.when(jj < HNCH // 2 - 1)
        def _():
            pltpu.async_copy(t_hbm.at[sidx.at[j0 + 2]], rows0, gsem0)

        pltpu.make_async_copy(t_hbm.at[sidx.at[j0 + 1]], rows1, gsem1).wait()
        pltpu.sync_copy(rows1, acc.at[didx.at[j0 + 1]], add=True)
        return carry

    for half in range(2):
        if half == 1:
            pltpu.sync_copy(src_hbm.at[wid, pl.ds(HNCH, HNCH)], sidx)
            pltpu.sync_copy(dst_hbm.at[wid, pl.ds(HNCH, HNCH)], didx)
        pltpu.async_copy(t_hbm.at[sidx.at[0]], rows0, gsem0)
        lax.fori_loop(0, HNCH // 2, body, 0)

    plsc.subcore_barrier()
    pltpu.sync_copy(acc.at[zslice], out_hbm.at[cid, zslice])


def _tc_mm_body(h_ref, w0_ref, q_ref):
    q_ref[...] = jnp.dot(h_ref[...], w0_ref[...],
                         preferred_element_type=jnp.float32)


def _tc1_body(deg_ref, q_ref, csrc_ref, cdst_ref, t0_ref):
    deg_out = deg_ref[0, 0, :N, :] + deg_ref[1, 0, :N, :]
    deg_in = deg_ref[0, 1, :N, :] + deg_ref[1, 1, :N, :]
    c_src = lax.rsqrt(jnp.maximum(deg_out, 1.0))
    c_dst = lax.rsqrt(jnp.maximum(deg_in, 1.0))
    csrc_ref[...] = c_src
    cdst_ref[...] = c_dst
    t0_ref[:N, :] = q_ref[...] * c_src
    t0_ref[N:, :] = jnp.zeros((N_PAD - N, D), jnp.float32)


def _tc_mid_body(p_ref, cdst_ref, b_ref, csrc_ref, w_ref, t_ref):
    agg = (p_ref[0, :N] + p_ref[1, :N]) * cdst_ref[...]
    hnew = jnp.maximum(agg + b_ref[...], 0.0)
    t_ref[:N, :] = jnp.dot(hnew * csrc_ref[...], w_ref[...],
                           preferred_element_type=jnp.float32)
    t_ref[N:, :] = jnp.zeros((N_PAD - N, D), jnp.float32)


def _tc_post_body(p_ref, cdst_ref, b_ref, out_ref):
    agg = (p_ref[0, :N] + p_ref[1, :N]) * cdst_ref[...]
    out_ref[...] = jnp.maximum(agg + b_ref[...], 0.0)


_tc_mm = pl.pallas_call(
    _tc_mm_body,
    out_shape=jax.ShapeDtypeStruct((N, D), jnp.float32),
)

_tc1 = pl.pallas_call(
    _tc1_body,
    out_shape=(
        jax.ShapeDtypeStruct((N, 1), jnp.float32),
        jax.ShapeDtypeStruct((N, 1), jnp.float32),
        jax.ShapeDtypeStruct((N_PAD, D), jnp.float32),
    ),
)

_tc_mid = pl.pallas_call(
    _tc_mid_body,
    out_shape=jax.ShapeDtypeStruct((N_PAD, D), jnp.float32),
)

_tc_post = pl.pallas_call(
    _tc_post_body,
    out_shape=jax.ShapeDtypeStruct((N, D), jnp.float32),
)


def kernel(h, edge_index, W0, b0, W1, b1):
    pad = (jnp.arange(E_PAD - E, dtype=jnp.int32) % (N_PAD - N)) + N
    src = jnp.concatenate([edge_index[0].astype(jnp.int32), pad])
    dst = jnp.concatenate([edge_index[1].astype(jnp.int32), pad])
    src = src.reshape(NW, NCH, C)
    dst = dst.reshape(NW, NCH, C)
    ones_c = jnp.ones((C,), jnp.float32)
    zeros_pad = jnp.zeros((N_PAD,), jnp.float32)
    zeros_nd = jnp.zeros((N_PAD, D), jnp.float32)

    q0 = _tc_mm(h, W0)
    deg = _deg_kernel(src, dst, ones_c, zeros_pad)
    deg4 = deg.reshape(NC, 2, N_PAD, 1)
    c_src, c_dst, t0 = _tc1(deg4, q0)

    p0 = _conv_kernel(t0, src, dst, zeros_nd)
    t1 = _tc_mid(p0, c_dst, b0.reshape(1, D), c_src, W1)

    p1 = _conv_kernel(t1, src, dst, zeros_nd)
    return _tc_post(p1, c_dst, b1.reshape(1, D))
